# baseline (speedup 1.0000x reference)
import functools

import jax
import jax.numpy as jnp
from jax import lax
from jax.experimental import pallas as pl
from jax.experimental.pallas import tpu as pltpu
from jax.experimental.pallas import tpu_sc as plsc

N_NODES = 10000
N_EDGES = 320000
NODE_DIM = 128
EDGE_DIM = 16
N_ITERS = 3

NC = 2
NS = 16
NW = NC * NS
CH = 128
NJ = 79
NCHP = NW * NJ
E_PAD = NCHP * CH
N_PAD = 10240
NROW = N_PAD // NS
DEAD_ROW = 10200

SD = 64
SU = 49
GF = 56
PITCH = 129
MW = 32


def _sc_gather_body(s_hbm, d_hbm, src_hbm, dst_hbm, g_hbm,
                    src_v, dst_v, sr0, dr0, sr1, dr1, sr2, dr2, sr3, dr3,
                    gtp, gt, sem_s0, sem_d0, sem_s1, sem_d1,
                    sem_s2, sem_d2, sem_s3, sem_d3):
    wid = lax.axis_index("s") * NC + lax.axis_index("c")
    pltpu.sync_copy(src_hbm.at[wid], src_v)
    pltpu.sync_copy(dst_hbm.at[wid], dst_v)
    iota = lax.iota(jnp.int32, 16)
    rows = [iota + 16 * b for b in range(4)]
    bufs = [(sr0, dr0, sem_s0, sem_d0), (sr1, dr1, sem_s1, sem_d1),
            (sr2, dr2, sem_s2, sem_d2), (sr3, dr3, sem_s3, sem_d3)]

    def issue(j, b):
        sr, dr, sem_s, sem_d = bufs[b]
        pltpu.async_copy(s_hbm.at[src_v.at[j]], sr, sem_s)
        pltpu.async_copy(d_hbm.at[dst_v.at[j]], dr, sem_d)

    def wait(j, b):
        sr, dr, sem_s, sem_d = bufs[b]
        pltpu.make_async_copy(s_hbm.at[src_v.at[j]], sr, sem_s).wait()
        pltpu.make_async_copy(d_hbm.at[dst_v.at[j]], dr, sem_d).wait()

    def work(j, b):
        sr, dr = bufs[b][0], bufs[b][1]

        def edges(e0, c):
            for u in range(4):
                e = e0 * 4 + u
                col = jnp.full((16,), 0, jnp.int32) + e
                for bb in range(4):
                    sl = pl.ds(16 * bb, 16)
                    v = sr[e, sl] + dr[e, sl]
                    plsc.store_scatter(gtp, [rows[bb], col], v)
            return c

        lax.fori_loop(0, CH // 4, edges, 0)
        for f in range(SU):
            for g in range(CH // 16):
                sl = pl.ds(g * 16, 16)
                gt[f, sl] = gtp[f, sl]
        pltpu.sync_copy(gt, g_hbm.at[wid * NJ + j])

    issue(0, 0)
    issue(1, 1)
    issue(2, 2)

    def quad(p, carry):
        j0 = 4 * p
        for u in range(4):
            issue(j0 + 3 + u, (3 + u) % 4)
            wait(j0 + u, u)
            work(j0 + u, u)
        return carry

    lax.fori_loop(0, NJ // 4, quad, 0)
    for u in range(NJ % 4):
        j = (NJ // 4) * 4 + u
        wait(j, u)
        work(j, u)


def _sc_scatter_body(uef_hbm, w_hbm, dst_hbm, zero_hbm, out_hbm,
                     dst_v, ue0, wv0, ue1, wv1, uwp, msg0, msg1, acc_sh,
                     sem_u0, sem_w0, sem_u1, sem_w1, sem_a):
    c = lax.axis_index("c")
    s = lax.axis_index("s")
    wid = s * NC + c

    pltpu.sync_copy(zero_hbm.at[pl.ds(s * NROW, NROW)],
                    acc_sh.at[pl.ds(s * NROW, NROW)])
    plsc.subcore_barrier()

    pltpu.sync_copy(dst_hbm.at[wid], dst_v)
    iota = lax.iota(jnp.int32, 16)
    zero16 = jnp.zeros((16,), jnp.float32)
    for f in range(EDGE_DIM + 1, MW):
        for g in range(CH // 16):
            uwp[f, pl.ds(g * 16, 16)] = zero16

    ins = [(ue0, wv0, sem_u0, sem_w0), (ue1, wv1, sem_u1, sem_w1)]
    msgs = [msg0, msg1]

    def issue(j, b):
        jj = wid * NJ + j
        ue, wv, sem_u, sem_w = ins[b]
        pltpu.async_copy(uef_hbm.at[jj], ue, sem_u)
        pltpu.async_copy(w_hbm.at[jj, 0], wv, sem_w)

    def wait(j, b):
        jj = wid * NJ + j
        ue, wv, sem_u, sem_w = ins[b]
        pltpu.make_async_copy(uef_hbm.at[jj], ue, sem_u).wait()
        pltpu.make_async_copy(w_hbm.at[jj, 0], wv, sem_w).wait()

    def work(j, b, first):
        ue, wv = ins[b][0], ins[b][1]
        msg_v = msgs[b]
        for g in range(CH // 16):
            sl = pl.ds(g * 16, 16)
            wvg = wv[sl]
            for f in range(EDGE_DIM):
                uwp[f, sl] = ue[f, sl] * wvg
            uwp[EDGE_DIM, sl] = wvg
        if not first:
            pltpu.make_async_copy(msg_v, acc_sh.at[dst_v.at[j]], sem_a).wait()
        def edges(e0, cc):
            for u in range(4):
                e = e0 * 4 + u
                col = jnp.full((16,), 0, jnp.int32) + e
                msg_v[e, pl.ds(0, 16)] = plsc.load_gather(uwp, [iota, col])
                msg_v[e, pl.ds(16, 16)] = plsc.load_gather(uwp, [iota + 16, col])
            return cc

        lax.fori_loop(0, CH // 4, edges, 0)
        pltpu.async_copy(msg_v, acc_sh.at[dst_v.at[j]], sem_a, add=True)

    issue(0, 0)
    issue(1, 1)
    issue(2, 2)

    def quad(p, carry):
        j0 = 4 * p
        for u in range(4):
            issue(j0 + 3 + u, (3 + u) % 4)
            wait(j0 + u, u)
            work(j0 + u, u)
        return carry

    lax.fori_loop(0, NJ // 4, quad, 0)
    for u in range(NJ % 4):
        j = (NJ // 4) * 4 + u
        wait(j, u)
        work(j, u)


def _sc_scatter_body(uef_hbm, w_hbm, dst_hbm, zero_hbm, out_hbm,
                     dst_v, ue0, wv0, ue1, wv1, uwp, msg0, msg1, acc_sh,
                     sem_u0, sem_w0, sem_u1, sem_w1, sem_a):
    c = lax.axis_index("c")
    s = lax.axis_index("s")
    wid = s * NC + c

    pltpu.sync_copy(zero_hbm.at[pl.ds(s * NROW, NROW)],
                    acc_sh.at[pl.ds(s * NROW, NROW)])
    plsc.subcore_barrier()

    pltpu.sync_copy(dst_hbm.at[wid], dst_v)
    iota = lax.iota(jnp.int32, 16)
    zero16 = jnp.zeros((16,), jnp.float32)
    for f in range(EDGE_DIM + 1, MW):
        for g in range(CH // 16):
            uwp[f, pl.ds(g * 16, 16)] = zero16

    ins = [(ue0, wv0, sem_u0, sem_w0), (ue1, wv1, sem_u1, sem_w1)]
    msgs = [msg0, msg1]

    def issue(j, b):
        jj = wid * NJ + j
        ue, wv, sem_u, sem_w = ins[b]
        pltpu.async_copy(uef_hbm.at[jj], ue, sem_u)
        pltpu.async_copy(w_hbm.at[jj, 0], wv, sem_w)

    def wait(j, b):
        jj = wid * NJ + j
        ue, wv, sem_u, sem_w = ins[b]
        pltpu.make_async_copy(uef_hbm.at[jj], ue, sem_u).wait()
        pltpu.make_async_copy(w_hbm.at[jj, 0], wv, sem_w).wait()

    def work(j, b, first):
        ue, wv = ins[b][0], ins[b][1]
        msg_v = msgs[b]
        for g in range(CH // 16):
            sl = pl.ds(g * 16, 16)
            wvg = wv[sl]
            for f in range(EDGE_DIM):
                uwp[f, sl] = ue[f, sl] * wvg
            uwp[EDGE_DIM, sl] = wvg
        if not first:
            pltpu.make_async_copy(msg_v, acc_sh.at[dst_v.at[j]], sem_a).wait()
        def edges(e0, cc):
            for u in range(4):
                e = e0 * 4 + u
                col = jnp.full((16,), 0, jnp.int32) + e
                msg_v[e, pl.ds(0, 16)] = plsc.load_gather(uwp, [iota, col])
                msg_v[e, pl.ds(16, 16)] = plsc.load_gather(uwp, [iota + 16, col])
            return cc

        lax.fori_loop(0, CH // 4, edges, 0)
        pltpu.async_copy(msg_v, acc_sh.at[dst_v.at[j]], sem_a, add=True)

    issue(0, 0)

    def pair(p, carry):
        j0 = 2 * p
        issue(j0 + 1, 1)
        wait(j0, 0)
        work(j0, 0, False)
        issue(j0 + 2, 0)
        wait(j0 + 1, 1)
        work(j0 + 1, 1, False)
        return carry

    issue(1, 1)
    wait(0, 0)
    work(0, 0, True)
    issue(2, 0)
    wait(1, 1)
    work(1, 1, True)

    def pair2(p, carry):
        j0 = 2 * p + 2
        issue(j0 + 1, 1)
        wait(j0, 0)
        work(j0, 0, False)
        issue(j0 + 2, 0)
        wait(j0 + 1, 1)
        work(j0 + 1, 1, False)
        return carry

    lax.fori_loop(0, (NJ - 3) // 2, pair2, 0)
    wait(NJ - 1, 0)
    work(NJ - 1, 0, False)
    pltpu.make_async_copy(msg0, acc_sh.at[dst_v.at[NJ - 1]], sem_a).wait()
    pltpu.make_async_copy(msg1, acc_sh.at[dst_v.at[NJ - 2]], sem_a).wait()
    plsc.subcore_barrier()

    pltpu.sync_copy(acc_sh.at[pl.ds(s * NROW, NROW)],
                    out_hbm.at[c, pl.ds(s * NROW, NROW)])


@functools.lru_cache(maxsize=None)
def _build_sc_kernels():
    mesh = plsc.VectorSubcoreMesh(core_axis_name="c", subcore_axis_name="s",
                                  num_cores=NC, num_subcores=NS)
    sc_gather = pl.kernel(
        _sc_gather_body,
        out_type=jax.ShapeDtypeStruct((NCHP, GF, CH), jnp.float32),
        mesh=mesh,
        compiler_params=pltpu.CompilerParams(use_tc_tiling_on_sc=False, needs_layout_passes=False),
        scratch_types=[
            pltpu.VMEM((NJ, CH), jnp.int32),
            pltpu.VMEM((NJ, CH), jnp.int32),
            pltpu.VMEM((CH, SD), jnp.float32),
            pltpu.VMEM((CH, SD), jnp.float32),
            pltpu.VMEM((CH, SD), jnp.float32),
            pltpu.VMEM((CH, SD), jnp.float32),
            pltpu.VMEM((CH, SD), jnp.float32),
            pltpu.VMEM((CH, SD), jnp.float32),
            pltpu.VMEM((CH, SD), jnp.float32),
            pltpu.VMEM((CH, SD), jnp.float32),
            pltpu.VMEM((GF, PITCH), jnp.float32),
            pltpu.VMEM((GF, CH), jnp.float32),
            pltpu.SemaphoreType.DMA,
            pltpu.SemaphoreType.DMA,
            pltpu.SemaphoreType.DMA,
            pltpu.SemaphoreType.DMA,
            pltpu.SemaphoreType.DMA,
            pltpu.SemaphoreType.DMA,
            pltpu.SemaphoreType.DMA,
            pltpu.SemaphoreType.DMA,
        ],
    )
    sc_scatter = pl.kernel(
        _sc_scatter_body,
        out_type=jax.ShapeDtypeStruct((NC, N_PAD, MW), jnp.float32),
        mesh=mesh,
        compiler_params=pltpu.CompilerParams(use_tc_tiling_on_sc=False, needs_layout_passes=False),
        scratch_types=[
            pltpu.VMEM((NJ, CH), jnp.int32),
            pltpu.VMEM((EDGE_DIM, CH), jnp.float32),
            pltpu.VMEM((CH,), jnp.float32),
            pltpu.VMEM((EDGE_DIM, CH), jnp.float32),
            pltpu.VMEM((CH,), jnp.float32),
            pltpu.VMEM((MW, PITCH), jnp.float32),
            pltpu.VMEM((CH, MW), jnp.float32),
            pltpu.VMEM((CH, MW), jnp.float32),
            pltpu.VMEM_SHARED((N_PAD, MW), jnp.float32),
            pltpu.SemaphoreType.DMA,
            pltpu.SemaphoreType.DMA,
            pltpu.SemaphoreType.DMA,
            pltpu.SemaphoreType.DMA,
            pltpu.SemaphoreType.DMA,
        ],
    )
    return sc_gather, sc_scatter


_CB = 8


def _edge_body(g_ref, ef_ref, whh_ref, bhh_ref, uef_ref, w_ref):
    for b in range(_CB):
        efb = ef_ref[b]
        ghb = lax.dot_general(whh_ref[...], efb, (((1,), (0,)), ((), ())),
                              preferred_element_type=jnp.float32) \
            + bhh_ref[...]
        gb = g_ref[b]
        r = jax.nn.sigmoid(gb[0:16] + ghb[0:16])
        z = jax.nn.sigmoid(gb[16:32] + ghb[16:32])
        n = jnp.tanh(gb[32:48] + r * ghb[32:48])
        uef_ref[b] = (1.0 - z) * n + z * efb
        w_ref[b, 0:1, :] = jnp.exp(gb[48:49] + ghb[48:49])
        w_ref[b, 1:8, :] = jnp.zeros((7, CH), jnp.float32)


def _node_body(a0_ref, a1_ref, nf_ref, wih_ref, whh_ref, bih_ref, bhh_ref,
               ws_ref, wd_ref, bs_ref, nfo_ref, s_ref, d_ref):
    a0 = a0_ref[...]
    a1 = a1_ref[...]
    nf = nf_ref[...]
    num = a0[:, :16] + a1[:, :16]
    den = a0[:, 16:17] + a1[:, 16:17]
    agg = jnp.where(den > 0.0, num / jnp.where(den > 0.0, den, 1.0), 0.0)
    gi = lax.dot_general(agg, wih_ref[...], (((1,), (1,)), ((), ())),
                         preferred_element_type=jnp.float32) + bih_ref[...]
    gh = lax.dot_general(nf, whh_ref[...], (((1,), (1,)), ((), ())),
                         preferred_element_type=jnp.float32) + bhh_ref[...]
    r = jax.nn.sigmoid(gi[:, 0:128] + gh[:, 0:128])
    z = jax.nn.sigmoid(gi[:, 128:256] + gh[:, 128:256])
    n = jnp.tanh(gi[:, 256:384] + r * gh[:, 256:384])
    nfo = (1.0 - z) * n + z * nf
    nfo_ref[...] = nfo
    s_ref[...] = lax.dot_general(nfo, ws_ref[...], (((1,), (1,)), ((), ())),
                                 preferred_element_type=jnp.float32) + bs_ref[...]
    d_ref[...] = lax.dot_general(nfo, wd_ref[...], (((1,), (1,)), ((), ())),
                                 preferred_element_type=jnp.float32)


def _proj_body(nf_ref, ws_ref, wd_ref, bs_ref, s_ref, d_ref):
    nf = nf_ref[...]
    s_ref[...] = lax.dot_general(nf, ws_ref[...], (((1,), (1,)), ((), ())),
                                 preferred_element_type=jnp.float32) + bs_ref[...]
    d_ref[...] = lax.dot_general(nf, wd_ref[...], (((1,), (1,)), ((), ())),
                                 preferred_element_type=jnp.float32)


_BN = 2000


def _full(shape):
    return pl.BlockSpec(shape, lambda i: (0,) * len(shape))


def _rows(shape):
    return pl.BlockSpec(shape, lambda i: (i,) + (0,) * (len(shape) - 1))


_edge_pass = pl.pallas_call(
    _edge_body,
    grid=(NCHP // _CB,),
    in_specs=[
        _rows((_CB, GF, CH)),
        _rows((_CB, EDGE_DIM, CH)),
        _full((49, EDGE_DIM)),
        _full((49, CH)),
    ],
    out_specs=[_rows((_CB, EDGE_DIM, CH)), _rows((_CB, 8, CH))],
    out_shape=[
        jax.ShapeDtypeStruct((NCHP, EDGE_DIM, CH), jnp.float32),
        jax.ShapeDtypeStruct((NCHP, 8, CH), jnp.float32),
    ],
)

_node_pass = pl.pallas_call(
    _node_body,
    grid=(N_NODES // _BN,),
    in_specs=[
        _rows((_BN, MW)),
        _rows((_BN, MW)),
        _rows((_BN, NODE_DIM)),
        _full((3 * NODE_DIM, EDGE_DIM)),
        _full((3 * NODE_DIM, NODE_DIM)),
        _full((1, 3 * NODE_DIM)),
        _full((1, 3 * NODE_DIM)),
        _full((SD, NODE_DIM)),
        _full((SD, NODE_DIM)),
        _full((1, SD)),
    ],
    out_specs=[_rows((_BN, NODE_DIM)), _rows((_BN, SD)), _rows((_BN, SD))],
    out_shape=[
        jax.ShapeDtypeStruct((N_NODES, NODE_DIM), jnp.float32),
        jax.ShapeDtypeStruct((N_NODES, SD), jnp.float32),
        jax.ShapeDtypeStruct((N_NODES, SD), jnp.float32),
    ],
)

_proj_pass = pl.pallas_call(
    _proj_body,
    grid=(N_NODES // _BN,),
    in_specs=[
        _rows((_BN, NODE_DIM)),
        _full((SD, NODE_DIM)),
        _full((SD, NODE_DIM)),
        _full((1, SD)),
    ],
    out_specs=[_rows((_BN, SD)), _rows((_BN, SD))],
    out_shape=[
        jax.ShapeDtypeStruct((N_NODES, SD), jnp.float32),
        jax.ShapeDtypeStruct((N_NODES, SD), jnp.float32),
    ],
)


def kernel(nf, ef, edge_index, W_ih_e, W_hh_e, b_ih_e, b_hh_e,
           W_ih_n, W_hh_n, b_ih_n, b_hh_n, W_attn, b_attn):
    ws = jnp.concatenate(
        [W_ih_e[:, :NODE_DIM], W_attn[:, :NODE_DIM],
         jnp.zeros((SD - SU, NODE_DIM), jnp.float32)], axis=0)
    wd = jnp.concatenate(
        [W_ih_e[:, NODE_DIM:], W_attn[:, NODE_DIM:2 * NODE_DIM],
         jnp.zeros((SD - SU, NODE_DIM), jnp.float32)], axis=0)
    bs = jnp.concatenate(
        [b_ih_e, b_attn, jnp.zeros((SD - SU,), jnp.float32)])[None, :]
    whh_ext = jnp.concatenate([W_hh_e, W_attn[:, 2 * NODE_DIM:]], axis=0)
    bhh_bc = jnp.broadcast_to(
        jnp.concatenate([b_hh_e, jnp.zeros((1,), jnp.float32)])[:, None],
        (49, CH))
    bih_n = b_ih_n[None, :]
    bhh_n = b_hh_n[None, :]

    pad_e = E_PAD - N_EDGES
    src_w = jnp.pad(edge_index[0], (0, pad_e)).reshape(NW, NJ, CH)
    dst_g = jnp.pad(edge_index[1], (0, pad_e)).reshape(NW, NJ, CH)
    dst_w = jnp.pad(edge_index[1], (0, pad_e),
                    constant_values=DEAD_ROW).reshape(NW, NJ, CH)
    zeros_acc = jnp.zeros((N_PAD, MW), jnp.float32)
    ef_t = jnp.pad(ef, ((0, pad_e), (0, 0))).reshape(
        NCHP, CH, EDGE_DIM).transpose(0, 2, 1)

    sc_gather, sc_scatter = _build_sc_kernels()

    s_t, d_t = _proj_pass(nf, ws, wd, bs)
    for _ in range(N_ITERS):
        g_t = sc_gather(s_t, d_t, src_w, dst_g)
        uef_t, w_t = _edge_pass(g_t, ef_t, whh_ext, bhh_bc)
        acc = sc_scatter(uef_t, w_t, dst_w, zeros_acc)
        nf, s_t, d_t = _node_pass(acc[0, :N_NODES], acc[1, :N_NODES], nf,
                                  W_ih_n, W_hh_n, bih_n, bhh_n, ws, wd, bs)
        ef_t = uef_t
    ef_out = ef_t.transpose(0, 2, 1).reshape(E_PAD, EDGE_DIM)[:N_EDGES]
    return (nf, ef_out)


# speedup vs baseline: 1.0001x; 1.0001x over previous
import functools

import jax
import jax.numpy as jnp
from jax import lax
from jax.experimental import pallas as pl
from jax.experimental.pallas import tpu as pltpu
from jax.experimental.pallas import tpu_sc as plsc

N_NODES = 10000
N_EDGES = 320000
NODE_DIM = 128
EDGE_DIM = 16
N_ITERS = 3

NC = 2
NS = 16
NW = NC * NS
CH = 128
NJ = 79
NCHP = NW * NJ
E_PAD = NCHP * CH
N_PAD = 10240
NROW = N_PAD // NS
DEAD_ROW = 10200

SD = 64
SU = 49
GF = 56
PITCH = 129
MW = 32


def _sc_gather_body(s_hbm, d_hbm, src_hbm, dst_hbm, g_hbm,
                    src_v, dst_v, sr0, dr0, sr1, dr1, sr2, dr2, sr3, dr3,
                    gtp, gt, sem_s0, sem_d0, sem_s1, sem_d1,
                    sem_s2, sem_d2, sem_s3, sem_d3):
    wid = lax.axis_index("s") * NC + lax.axis_index("c")
    pltpu.sync_copy(src_hbm.at[wid], src_v)
    pltpu.sync_copy(dst_hbm.at[wid], dst_v)
    iota = lax.iota(jnp.int32, 16)
    rows = [iota + 16 * b for b in range(4)]
    bufs = [(sr0, dr0, sem_s0, sem_d0), (sr1, dr1, sem_s1, sem_d1),
            (sr2, dr2, sem_s2, sem_d2), (sr3, dr3, sem_s3, sem_d3)]

    def issue(j, b):
        sr, dr, sem_s, sem_d = bufs[b]
        pltpu.async_copy(s_hbm.at[src_v.at[j]], sr, sem_s)
        pltpu.async_copy(d_hbm.at[dst_v.at[j]], dr, sem_d)

    def wait(j, b):
        sr, dr, sem_s, sem_d = bufs[b]
        pltpu.make_async_copy(s_hbm.at[src_v.at[j]], sr, sem_s).wait()
        pltpu.make_async_copy(d_hbm.at[dst_v.at[j]], dr, sem_d).wait()

    def work(j, b):
        sr, dr = bufs[b][0], bufs[b][1]

        def edges(e0, c):
            for u in range(4):
                e = e0 * 4 + u
                col = jnp.full((16,), 0, jnp.int32) + e
                for bb in range(4):
                    sl = pl.ds(16 * bb, 16)
                    v = sr[e, sl] + dr[e, sl]
                    plsc.store_scatter(gtp, [rows[bb], col], v)
            return c

        lax.fori_loop(0, CH // 4, edges, 0)
        for f in range(SU):
            for g in range(CH // 16):
                sl = pl.ds(g * 16, 16)
                gt[f, sl] = gtp[f, sl]
        pltpu.sync_copy(gt, g_hbm.at[wid * NJ + j])

    issue(0, 0)
    issue(1, 1)
    issue(2, 2)

    def quad(p, carry):
        j0 = 4 * p
        for u in range(4):
            issue(j0 + 3 + u, (3 + u) % 4)
            wait(j0 + u, u)
            work(j0 + u, u)
        return carry

    lax.fori_loop(0, NJ // 4, quad, 0)
    for u in range(NJ % 4):
        j = (NJ // 4) * 4 + u
        wait(j, u)
        work(j, u)


def _sc_scatter_body(uef_hbm, w_hbm, dst_hbm, zero_hbm, out_hbm,
                     dst_v, ue0, wv0, ue1, wv1, uwp, msg0, msg1, acc_sh,
                     sem_u0, sem_w0, sem_u1, sem_w1, sem_a):
    c = lax.axis_index("c")
    s = lax.axis_index("s")
    wid = s * NC + c

    pltpu.sync_copy(zero_hbm.at[pl.ds(s * NROW, NROW)],
                    acc_sh.at[pl.ds(s * NROW, NROW)])
    plsc.subcore_barrier()

    pltpu.sync_copy(dst_hbm.at[wid], dst_v)
    iota = lax.iota(jnp.int32, 16)
    zero16 = jnp.zeros((16,), jnp.float32)
    for f in range(EDGE_DIM + 1, MW):
        for g in range(CH // 16):
            uwp[f, pl.ds(g * 16, 16)] = zero16

    ins = [(ue0, wv0, sem_u0, sem_w0), (ue1, wv1, sem_u1, sem_w1)]
    msgs = [msg0, msg1]

    def issue(j, b):
        jj = wid * NJ + j
        ue, wv, sem_u, sem_w = ins[b]
        pltpu.async_copy(uef_hbm.at[jj], ue, sem_u)
        pltpu.async_copy(w_hbm.at[jj, 0], wv, sem_w)

    def wait(j, b):
        jj = wid * NJ + j
        ue, wv, sem_u, sem_w = ins[b]
        pltpu.make_async_copy(uef_hbm.at[jj], ue, sem_u).wait()
        pltpu.make_async_copy(w_hbm.at[jj, 0], wv, sem_w).wait()

    def work(j, b, first):
        ue, wv = ins[b][0], ins[b][1]
        msg_v = msgs[b]
        for g in range(CH // 16):
            sl = pl.ds(g * 16, 16)
            wvg = wv[sl]
            for f in range(EDGE_DIM):
                uwp[f, sl] = ue[f, sl] * wvg
            uwp[EDGE_DIM, sl] = wvg
        if not first:
            pltpu.make_async_copy(msg_v, acc_sh.at[dst_v.at[j]], sem_a).wait()
        def edges(e0, cc):
            for u in range(4):
                e = e0 * 4 + u
                col = jnp.full((16,), 0, jnp.int32) + e
                msg_v[e, pl.ds(0, 16)] = plsc.load_gather(uwp, [iota, col])
                msg_v[e, pl.ds(16, 16)] = plsc.load_gather(uwp, [iota + 16, col])
            return cc

        lax.fori_loop(0, CH // 4, edges, 0)
        pltpu.async_copy(msg_v, acc_sh.at[dst_v.at[j]], sem_a, add=True)

    issue(0, 0)
    issue(1, 1)
    issue(2, 2)

    def quad(p, carry):
        j0 = 4 * p
        for u in range(4):
            issue(j0 + 3 + u, (3 + u) % 4)
            wait(j0 + u, u)
            work(j0 + u, u)
        return carry

    lax.fori_loop(0, NJ // 4, quad, 0)
    for u in range(NJ % 4):
        j = (NJ // 4) * 4 + u
        wait(j, u)
        work(j, u)


def _sc_scatter_body(uef_hbm, w_hbm, dst_hbm, zero_hbm, out_hbm,
                     dst_v, ue0, wv0, ue1, wv1, uwp, msg0, msg1, acc_sh,
                     sem_u0, sem_w0, sem_u1, sem_w1, sem_a):
    c = lax.axis_index("c")
    s = lax.axis_index("s")
    wid = s * NC + c

    pltpu.sync_copy(zero_hbm.at[pl.ds(s * NROW, NROW)],
                    acc_sh.at[pl.ds(s * NROW, NROW)])
    plsc.subcore_barrier()

    pltpu.sync_copy(dst_hbm.at[wid], dst_v)
    iota = lax.iota(jnp.int32, 16)
    zero16 = jnp.zeros((16,), jnp.float32)
    for f in range(EDGE_DIM + 1, MW):
        for g in range(CH // 16):
            uwp[f, pl.ds(g * 16, 16)] = zero16

    ins = [(ue0, wv0, sem_u0, sem_w0), (ue1, wv1, sem_u1, sem_w1)]
    msgs = [msg0, msg1]

    def issue(j, b):
        jj = wid * NJ + j
        ue, wv, sem_u, sem_w = ins[b]
        pltpu.async_copy(uef_hbm.at[jj], ue, sem_u)
        pltpu.async_copy(w_hbm.at[jj, 0], wv, sem_w)

    def wait(j, b):
        jj = wid * NJ + j
        ue, wv, sem_u, sem_w = ins[b]
        pltpu.make_async_copy(uef_hbm.at[jj], ue, sem_u).wait()
        pltpu.make_async_copy(w_hbm.at[jj, 0], wv, sem_w).wait()

    def work(j, b, first):
        ue, wv = ins[b][0], ins[b][1]
        msg_v = msgs[b]
        for g in range(CH // 16):
            sl = pl.ds(g * 16, 16)
            wvg = wv[sl]
            for f in range(EDGE_DIM):
                uwp[f, sl] = ue[f, sl] * wvg
            uwp[EDGE_DIM, sl] = wvg
        if not first:
            pltpu.make_async_copy(msg_v, acc_sh.at[dst_v.at[j]], sem_a).wait()
        def edges(e0, cc):
            for u in range(4):
                e = e0 * 4 + u
                col = jnp.full((16,), 0, jnp.int32) + e
                msg_v[e, pl.ds(0, 16)] = plsc.load_gather(uwp, [iota, col])
                msg_v[e, pl.ds(16, 16)] = plsc.load_gather(uwp, [iota + 16, col])
            return cc

        lax.fori_loop(0, CH // 4, edges, 0)
        pltpu.async_copy(msg_v, acc_sh.at[dst_v.at[j]], sem_a, add=True)

    issue(0, 0)

    issue(1, 1)
    wait(0, 0)
    work(0, 0, True)
    issue(2, 0)
    wait(1, 1)
    work(1, 1, True)

    def pair2(p, carry):
        j0 = 2 * p + 2
        issue(j0 + 1, 1)
        wait(j0, 0)
        work(j0, 0, False)
        issue(j0 + 2, 0)
        wait(j0 + 1, 1)
        work(j0 + 1, 1, False)
        return carry

    lax.fori_loop(0, (NJ - 3) // 2, pair2, 0)
    wait(NJ - 1, 0)
    work(NJ - 1, 0, False)
    pltpu.make_async_copy(msg0, acc_sh.at[dst_v.at[NJ - 1]], sem_a).wait()
    pltpu.make_async_copy(msg1, acc_sh.at[dst_v.at[NJ - 2]], sem_a).wait()
    plsc.subcore_barrier()

    pltpu.sync_copy(acc_sh.at[pl.ds(s * NROW, NROW)],
                    out_hbm.at[c, pl.ds(s * NROW, NROW)])


@functools.lru_cache(maxsize=None)
def _build_sc_kernels():
    mesh = plsc.VectorSubcoreMesh(core_axis_name="c", subcore_axis_name="s",
                                  num_cores=NC, num_subcores=NS)
    sc_gather = pl.kernel(
        _sc_gather_body,
        out_type=jax.ShapeDtypeStruct((NCHP, GF, CH), jnp.float32),
        mesh=mesh,
        compiler_params=pltpu.CompilerParams(use_tc_tiling_on_sc=False, needs_layout_passes=False),
        scratch_types=[
            pltpu.VMEM((NJ, CH), jnp.int32),
            pltpu.VMEM((NJ, CH), jnp.int32),
            pltpu.VMEM((CH, SD), jnp.float32),
            pltpu.VMEM((CH, SD), jnp.float32),
            pltpu.VMEM((CH, SD), jnp.float32),
            pltpu.VMEM((CH, SD), jnp.float32),
            pltpu.VMEM((CH, SD), jnp.float32),
            pltpu.VMEM((CH, SD), jnp.float32),
            pltpu.VMEM((CH, SD), jnp.float32),
            pltpu.VMEM((CH, SD), jnp.float32),
            pltpu.VMEM((GF, PITCH), jnp.float32),
            pltpu.VMEM((GF, CH), jnp.float32),
            pltpu.SemaphoreType.DMA,
            pltpu.SemaphoreType.DMA,
            pltpu.SemaphoreType.DMA,
            pltpu.SemaphoreType.DMA,
            pltpu.SemaphoreType.DMA,
            pltpu.SemaphoreType.DMA,
            pltpu.SemaphoreType.DMA,
            pltpu.SemaphoreType.DMA,
        ],
    )
    sc_scatter = pl.kernel(
        _sc_scatter_body,
        out_type=jax.ShapeDtypeStruct((NC, N_PAD, MW), jnp.float32),
        mesh=mesh,
        compiler_params=pltpu.CompilerParams(use_tc_tiling_on_sc=False, needs_layout_passes=False),
        scratch_types=[
            pltpu.VMEM((NJ, CH), jnp.int32),
            pltpu.VMEM((EDGE_DIM, CH), jnp.float32),
            pltpu.VMEM((CH,), jnp.float32),
            pltpu.VMEM((EDGE_DIM, CH), jnp.float32),
            pltpu.VMEM((CH,), jnp.float32),
            pltpu.VMEM((MW, PITCH), jnp.float32),
            pltpu.VMEM((CH, MW), jnp.float32),
            pltpu.VMEM((CH, MW), jnp.float32),
            pltpu.VMEM_SHARED((N_PAD, MW), jnp.float32),
            pltpu.SemaphoreType.DMA,
            pltpu.SemaphoreType.DMA,
            pltpu.SemaphoreType.DMA,
            pltpu.SemaphoreType.DMA,
            pltpu.SemaphoreType.DMA,
        ],
    )
    return sc_gather, sc_scatter


_CB = 8


def _edge_body(g_ref, ef_ref, whh_ref, bhh_ref, uef_ref, w_ref):
    for b in range(_CB):
        efb = ef_ref[b]
        ghb = lax.dot_general(whh_ref[...], efb, (((1,), (0,)), ((), ())),
                              preferred_element_type=jnp.float32) \
            + bhh_ref[...]
        gb = g_ref[b]
        r = jax.nn.sigmoid(gb[0:16] + ghb[0:16])
        z = jax.nn.sigmoid(gb[16:32] + ghb[16:32])
        n = jnp.tanh(gb[32:48] + r * ghb[32:48])
        uef_ref[b] = (1.0 - z) * n + z * efb
        w_ref[b, 0:1, :] = jnp.exp(gb[48:49] + ghb[48:49])
        w_ref[b, 1:8, :] = jnp.zeros((7, CH), jnp.float32)


def _node_body(a0_ref, a1_ref, nf_ref, wih_ref, whh_ref, bih_ref, bhh_ref,
               ws_ref, wd_ref, bs_ref, nfo_ref, s_ref, d_ref):
    a0 = a0_ref[...]
    a1 = a1_ref[...]
    nf = nf_ref[...]
    num = a0[:, :16] + a1[:, :16]
    den = a0[:, 16:17] + a1[:, 16:17]
    agg = jnp.where(den > 0.0, num / jnp.where(den > 0.0, den, 1.0), 0.0)
    gi = lax.dot_general(agg, wih_ref[...], (((1,), (1,)), ((), ())),
                         preferred_element_type=jnp.float32) + bih_ref[...]
    gh = lax.dot_general(nf, whh_ref[...], (((1,), (1,)), ((), ())),
                         preferred_element_type=jnp.float32) + bhh_ref[...]
    r = jax.nn.sigmoid(gi[:, 0:128] + gh[:, 0:128])
    z = jax.nn.sigmoid(gi[:, 128:256] + gh[:, 128:256])
    n = jnp.tanh(gi[:, 256:384] + r * gh[:, 256:384])
    nfo = (1.0 - z) * n + z * nf
    nfo_ref[...] = nfo
    s_ref[...] = lax.dot_general(nfo, ws_ref[...], (((1,), (1,)), ((), ())),
                                 preferred_element_type=jnp.float32) + bs_ref[...]
    d_ref[...] = lax.dot_general(nfo, wd_ref[...], (((1,), (1,)), ((), ())),
                                 preferred_element_type=jnp.float32)


def _proj_body(nf_ref, ws_ref, wd_ref, bs_ref, s_ref, d_ref):
    nf = nf_ref[...]
    s_ref[...] = lax.dot_general(nf, ws_ref[...], (((1,), (1,)), ((), ())),
                                 preferred_element_type=jnp.float32) + bs_ref[...]
    d_ref[...] = lax.dot_general(nf, wd_ref[...], (((1,), (1,)), ((), ())),
                                 preferred_element_type=jnp.float32)


_BN = 2000


def _full(shape):
    return pl.BlockSpec(shape, lambda i: (0,) * len(shape))


def _rows(shape):
    return pl.BlockSpec(shape, lambda i: (i,) + (0,) * (len(shape) - 1))


_edge_pass = pl.pallas_call(
    _edge_body,
    grid=(NCHP // _CB,),
    in_specs=[
        _rows((_CB, GF, CH)),
        _rows((_CB, EDGE_DIM, CH)),
        _full((49, EDGE_DIM)),
        _full((49, CH)),
    ],
    out_specs=[_rows((_CB, EDGE_DIM, CH)), _rows((_CB, 8, CH))],
    out_shape=[
        jax.ShapeDtypeStruct((NCHP, EDGE_DIM, CH), jnp.float32),
        jax.ShapeDtypeStruct((NCHP, 8, CH), jnp.float32),
    ],
)

_node_pass = pl.pallas_call(
    _node_body,
    grid=(N_NODES // _BN,),
    in_specs=[
        _rows((_BN, MW)),
        _rows((_BN, MW)),
        _rows((_BN, NODE_DIM)),
        _full((3 * NODE_DIM, EDGE_DIM)),
        _full((3 * NODE_DIM, NODE_DIM)),
        _full((1, 3 * NODE_DIM)),
        _full((1, 3 * NODE_DIM)),
        _full((SD, NODE_DIM)),
        _full((SD, NODE_DIM)),
        _full((1, SD)),
    ],
    out_specs=[_rows((_BN, NODE_DIM)), _rows((_BN, SD)), _rows((_BN, SD))],
    out_shape=[
        jax.ShapeDtypeStruct((N_NODES, NODE_DIM), jnp.float32),
        jax.ShapeDtypeStruct((N_NODES, SD), jnp.float32),
        jax.ShapeDtypeStruct((N_NODES, SD), jnp.float32),
    ],
)

_proj_pass = pl.pallas_call(
    _proj_body,
    grid=(N_NODES // _BN,),
    in_specs=[
        _rows((_BN, NODE_DIM)),
        _full((SD, NODE_DIM)),
        _full((SD, NODE_DIM)),
        _full((1, SD)),
    ],
    out_specs=[_rows((_BN, SD)), _rows((_BN, SD))],
    out_shape=[
        jax.ShapeDtypeStruct((N_NODES, SD), jnp.float32),
        jax.ShapeDtypeStruct((N_NODES, SD), jnp.float32),
    ],
)


def kernel(nf, ef, edge_index, W_ih_e, W_hh_e, b_ih_e, b_hh_e,
           W_ih_n, W_hh_n, b_ih_n, b_hh_n, W_attn, b_attn):
    ws = jnp.concatenate(
        [W_ih_e[:, :NODE_DIM], W_attn[:, :NODE_DIM],
         jnp.zeros((SD - SU, NODE_DIM), jnp.float32)], axis=0)
    wd = jnp.concatenate(
        [W_ih_e[:, NODE_DIM:], W_attn[:, NODE_DIM:2 * NODE_DIM],
         jnp.zeros((SD - SU, NODE_DIM), jnp.float32)], axis=0)
    bs = jnp.concatenate(
        [b_ih_e, b_attn, jnp.zeros((SD - SU,), jnp.float32)])[None, :]
    whh_ext = jnp.concatenate([W_hh_e, W_attn[:, 2 * NODE_DIM:]], axis=0)
    bhh_bc = jnp.broadcast_to(
        jnp.concatenate([b_hh_e, jnp.zeros((1,), jnp.float32)])[:, None],
        (49, CH))
    bih_n = b_ih_n[None, :]
    bhh_n = b_hh_n[None, :]

    pad_e = E_PAD - N_EDGES
    src_w = jnp.pad(edge_index[0], (0, pad_e)).reshape(NW, NJ, CH)
    dst_g = jnp.pad(edge_index[1], (0, pad_e)).reshape(NW, NJ, CH)
    dst_w = jnp.pad(edge_index[1], (0, pad_e),
                    constant_values=DEAD_ROW).reshape(NW, NJ, CH)
    zeros_acc = jnp.zeros((N_PAD, MW), jnp.float32)
    ef_t = jnp.pad(ef, ((0, pad_e), (0, 0))).reshape(
        NCHP, CH, EDGE_DIM).transpose(0, 2, 1)

    sc_gather, sc_scatter = _build_sc_kernels()

    s_t, d_t = _proj_pass(nf, ws, wd, bs)
    for _ in range(N_ITERS):
        g_t = sc_gather(s_t, d_t, src_w, dst_g)
        uef_t, w_t = _edge_pass(g_t, ef_t, whh_ext, bhh_bc)
        acc = sc_scatter(uef_t, w_t, dst_w, zeros_acc)
        nf, s_t, d_t = _node_pass(acc[0, :N_NODES], acc[1, :N_NODES], nf,
                                  W_ih_n, W_hh_n, bih_n, bhh_n, ws, wd, bs)
        ef_t = uef_t
    ef_out = ef_t.transpose(0, 2, 1).reshape(E_PAD, EDGE_DIM)[:N_EDGES]
    return (nf, ef_out)
